# baseline (speedup 1.0000x reference)
import functools

import jax
import jax.numpy as jnp
from jax import lax
from jax.experimental import pallas as pl
from jax.experimental.pallas import tpu as pltpu
from jax.experimental.pallas import tpu_sc as plsc

B = 16384
D = 32
NC = 2
NS = 16
NW = NC * NS
BPW = B // NW
CH = 128
NCH = BPW // CH


def _sc_gather(uid_hbm, iid_hbm, gu_hbm, gi_hbm, mu_hbm, mi_hbm,
               gu_out, gi_out, mu_out, mi_out,
               uidx_v, iidx_v, gu_v, gi_v, mu_v, mi_v, sem):
    wid = lax.axis_index("s") * NC + lax.axis_index("c")
    pltpu.sync_copy(uid_hbm.at[pl.ds(wid * NCH, NCH)], uidx_v)
    pltpu.sync_copy(iid_hbm.at[pl.ds(wid * NCH, NCH)], iidx_v)
    copies = []
    for c in range(NCH):
        sl = pl.ds(c * CH, CH)
        copies.append(pltpu.async_copy(gu_hbm.at[uidx_v.at[c]], gu_v.at[sl], sem))
        copies.append(pltpu.async_copy(gi_hbm.at[iidx_v.at[c]], gi_v.at[sl], sem))
        copies.append(pltpu.async_copy(mu_hbm.at[uidx_v.at[c]], mu_v.at[sl], sem))
        copies.append(pltpu.async_copy(mi_hbm.at[iidx_v.at[c]], mi_v.at[sl], sem))
    for cp in copies:
        cp.wait()
    rows = pl.ds(wid * BPW, BPW)
    pltpu.sync_copy(gu_v, gu_out.at[rows])
    pltpu.sync_copy(gi_v, gi_out.at[rows])
    pltpu.sync_copy(mu_v, mu_out.at[rows])
    pltpu.sync_copy(mi_v, mi_out.at[rows])


@functools.cache
def _sc_gather_call():
    return functools.partial(
        pl.kernel,
        mesh=plsc.VectorSubcoreMesh(core_axis_name="c", subcore_axis_name="s"),
        out_type=[jax.ShapeDtypeStruct((B, D), jnp.float32)] * 4,
        scratch_types=[
            pltpu.VMEM((NCH, CH), jnp.int32),
            pltpu.VMEM((NCH, CH), jnp.int32),
            pltpu.VMEM((BPW, D), jnp.float32),
            pltpu.VMEM((BPW, D), jnp.float32),
            pltpu.VMEM((BPW, D), jnp.float32),
            pltpu.VMEM((BPW, D), jnp.float32),
            pltpu.SemaphoreType.DMA,
        ],
        compiler_params=pltpu.CompilerParams(use_tc_tiling_on_sc=False),
    )(_sc_gather)


BT = 2048


def _tc_body(gu, gi, mu, mi, w1u, w1i, b1, w2, b2, w3, b3, wog, woh, bo, out):
    h = (jnp.dot(mu[...], w1u[...], preferred_element_type=jnp.float32)
         + jnp.dot(mi[...], w1i[...], preferred_element_type=jnp.float32)
         + b1[...])
    h = jnp.maximum(h, 0.0)
    h = jnp.maximum(jnp.dot(h, w2[...], preferred_element_type=jnp.float32) + b2[...], 0.0)
    h = jnp.maximum(jnp.dot(h, w3[...], preferred_element_type=jnp.float32) + b3[...], 0.0)
    g = gu[...] * gi[...]
    out[...] = (jnp.sum(g * wog[...], axis=1) + jnp.sum(h * woh[...], axis=1)
                + bo[0, 0])


def _tc_mlp(gu, gi, mu, mi, w1u, w1i, b1, w2, b2, w3, b3, wog, woh, bo):
    rows = lambda: pl.BlockSpec((BT, D), lambda i: (i, 0))
    full = lambda a: pl.BlockSpec(a.shape, lambda i: (0,) * a.ndim)
    return pl.pallas_call(
        _tc_body,
        grid=(B // BT,),
        in_specs=[rows(), rows(), rows(), rows(),
                  full(w1u), full(w1i), full(b1), full(w2), full(b2),
                  full(w3), full(b3), full(wog), full(woh), full(bo)],
        out_specs=pl.BlockSpec((BT,), lambda i: (i,)),
        out_shape=jax.ShapeDtypeStruct((B,), jnp.float32),
    )(gu, gi, mu, mi, w1u, w1i, b1, w2, b2, w3, b3, wog, woh, bo)


def kernel(user_ids, item_ids, gmf_user_w, gmf_item_w, mlp_user_w, mlp_item_w,
           W1, b1, W2, b2, W3, b3, Wo, bo):
    uid2 = user_ids.astype(jnp.int32).reshape(B // CH, CH)
    iid2 = item_ids.astype(jnp.int32).reshape(B // CH, CH)
    gu, gi, mu, mi = _sc_gather_call()(uid2, iid2, gmf_user_w, gmf_item_w,
                                       mlp_user_w, mlp_item_w)
    w1u = W1[:, :D].T
    w1i = W1[:, D:].T
    out = _tc_mlp(gu, gi, mu, mi,
                  w1u, w1i, b1.reshape(1, -1),
                  W2.T, b2.reshape(1, -1),
                  W3.T, b3.reshape(1, -1),
                  Wo[:, :D], Wo[:, D:], bo.reshape(1, 1))
    return out
